# Initial kernel scaffold; baseline (speedup 1.0000x reference)
import jax
import jax.numpy as jnp
from jax.experimental import pallas as pl


def kernel(atom_type, edge_index, edge_type, batch, emb, W0, root0, b0, W1, root1, b1):
    raise NotImplementedError("write your pallas kernel here")


# speedup vs baseline: 11.0131x; 11.0131x over previous
import functools

import jax
import jax.numpy as jnp
from jax import lax
from jax.experimental import pallas as pl
from jax.experimental.pallas import tpu as pltpu
from jax.experimental.pallas import tpu_sc as plsc

N = 10000
E = 320000
D = 128
R = 4
VOCAB = 100
B = 256

NS = 16
SLAB = 32
NQ = 4
CHUNK = 128
ECH = 2512
E_PAD = ECH * CHUNK
CPT = ECH // NS
ACC_R = 40960
RPT = ACC_R // NS
NH = 10240
BN = 400
NB = N // BN

_mesh = lambda: plsc.VectorSubcoreMesh(core_axis_name="c", subcore_axis_name="s")


def _sc_prep(e0, e1, e2, e3, atom2, dstadj2, ones_in, zcnt,
             h0, h1, h2, h3, cnt_out,
             cnt_sh, aidx, didx, rows, ones_v, sem):
  c = lax.axis_index("c")
  s = lax.axis_index("s")
  e_refs = (e0, e1, e2, e3)
  h_refs = (h0, h1, h2, h3)

  @pl.when(c == 1)
  def _():
    part = s % 4
    pltpu.sync_copy(atom2.at[part], aidx)
    for q in range(NQ):
      @pl.when(s // 4 == q)
      def _(q=q):
        def chunk(k, carry):
          base = part * 2560 + k * CHUNK
          pltpu.async_copy(e_refs[q].at[aidx.at[k]], rows, sem).wait()
          pltpu.sync_copy(rows, h_refs[q].at[pl.ds(base, CHUNK)])
          return carry
        lax.fori_loop(0, 20, chunk, 0)

  @pl.when(c == 0)
  def _():
    pltpu.sync_copy(ones_in, ones_v)
    pltpu.sync_copy(zcnt.at[pl.ds(s * RPT, RPT)], cnt_sh.at[pl.ds(s * RPT, RPT)])
    pltpu.sync_copy(dstadj2.at[s], didx)
    plsc.subcore_barrier()

    def chunk(k, carry):
      pltpu.sync_copy(ones_v, cnt_sh.at[didx.at[k]], add=True)
      return carry
    lax.fori_loop(0, CPT, chunk, 0)
    plsc.subcore_barrier()
    pltpu.sync_copy(cnt_sh.at[pl.ds(s * RPT, RPT)], cnt_out.at[pl.ds(s * RPT, RPT)])


def _sc_edge(h0, h1, h2, h3, src2, dstadj2, zacc,
             a0, a1, a2, a3,
             acc_sh, sidx, didx, rA, rB, semA, semB, semSA, semSB):
  c = lax.axis_index("c")
  s = lax.axis_index("s")
  h_refs = (h0, h1, h2, h3)
  a_refs = (a0, a1, a2, a3)

  pltpu.sync_copy(src2.at[s], sidx)
  pltpu.sync_copy(dstadj2.at[s], didx)

  for p in (0, 1):
    for cc in (0, 1):
      @pl.when(c == cc)
      def _(q=2 * cc + p):
        h = h_refs[q]
        aout = a_refs[q]
        pltpu.sync_copy(zacc.at[pl.ds(s * RPT, RPT)], acc_sh.at[pl.ds(s * RPT, RPT)])
        plsc.subcore_barrier()

        def pair(j, carry):
          k0 = 2 * j
          k1 = k0 + 1
          ga = pltpu.async_copy(h.at[sidx.at[k0]], rA, semA)
          gb = pltpu.async_copy(h.at[sidx.at[k1]], rB, semB)
          ga.wait()
          sa = pltpu.async_copy(rA, acc_sh.at[didx.at[k0]], semSA, add=True)
          gb.wait()
          sb = pltpu.async_copy(rB, acc_sh.at[didx.at[k1]], semSB, add=True)
          sa.wait()
          sb.wait()
          return carry
        lax.fori_loop(0, CPT // 2, pair, 0)

        k = CPT - 1
        pltpu.async_copy(h.at[sidx.at[k]], rA, semA).wait()
        pltpu.async_copy(rA, acc_sh.at[didx.at[k]], semSA, add=True).wait()
        plsc.subcore_barrier()
        pltpu.sync_copy(acc_sh.at[pl.ds(s * RPT, RPT)], aout.at[pl.ds(s * RPT, RPT)])


def _sc_prep_call(e_slabs, atom2, dstadj2):
  f32 = jnp.float32
  ones_in = jnp.ones((CHUNK, 16), f32)
  zcnt = jnp.zeros((ACC_R, 16), f32)
  fn = pl.kernel(
      _sc_prep,
      out_type=[jax.ShapeDtypeStruct((NH, SLAB), f32) for _ in range(NQ)]
      + [jax.ShapeDtypeStruct((ACC_R, 16), f32)],
      mesh=_mesh(),
      compiler_params=pltpu.CompilerParams(use_tc_tiling_on_sc=False),
      scratch_types=[
          pltpu.VMEM_SHARED((ACC_R, 16), f32),
          pltpu.VMEM((20, CHUNK), jnp.int32),
          pltpu.VMEM((CPT, CHUNK), jnp.int32),
          pltpu.VMEM((CHUNK, SLAB), f32),
          pltpu.VMEM((CHUNK, 16), f32),
          pltpu.SemaphoreType.DMA,
      ],
  )
  return fn(*e_slabs, atom2, dstadj2, ones_in, zcnt)


def _sc_edge_call(h_slabs, src2, dstadj2):
  f32 = jnp.float32
  zacc = jnp.zeros((ACC_R, SLAB), f32)
  fn = pl.kernel(
      _sc_edge,
      out_type=[jax.ShapeDtypeStruct((ACC_R, SLAB), f32) for _ in range(NQ)],
      mesh=_mesh(),
      compiler_params=pltpu.CompilerParams(use_tc_tiling_on_sc=False),
      scratch_types=[
          pltpu.VMEM_SHARED((ACC_R, SLAB), f32),
          pltpu.VMEM((CPT, CHUNK), jnp.int32),
          pltpu.VMEM((CPT, CHUNK), jnp.int32),
          pltpu.VMEM((CHUNK, SLAB), f32),
          pltpu.VMEM((CHUNK, SLAB), f32),
          pltpu.SemaphoreType.DMA,
          pltpu.SemaphoreType.DMA,
          pltpu.SemaphoreType.DMA,
          pltpu.SemaphoreType.DMA,
      ],
  )
  return fn(*h_slabs, src2, dstadj2, zacc)


def _tc_layer0(h0, h1, h2, h3, a0, a1, a2, a3, cnt, w, root, b,
               o0, o1, o2, o3, acc_s):
  r = pl.program_id(1)
  inv = 1.0 / jnp.maximum(cnt[:, 0:1], 1.0)
  accb = jnp.concatenate([a0[...], a1[...], a2[...], a3[...]], axis=1)
  contrib = jnp.dot(accb * inv, w[0], preferred_element_type=jnp.float32)

  @pl.when(r == 0)
  def _():
    hb = jnp.concatenate([h0[...], h1[...], h2[...], h3[...]], axis=1)
    acc_s[...] = jnp.dot(hb, root[...], preferred_element_type=jnp.float32) + b[...] + contrib

  @pl.when(r != 0)
  def _():
    acc_s[...] = acc_s[...] + contrib

  @pl.when(r == R - 1)
  def _():
    res = jnp.tanh(acc_s[...])
    outs = (o0, o1, o2, o3)
    for q in range(NQ):
      outs[q][...] = res[:, q * SLAB:(q + 1) * SLAB]


def _tc_layer1(h0, h1, h2, h3, a0, a1, a2, a3, cnt, w, root, b, batch2,
               final, acc_s, psum_s, pcnt_s):
  i = pl.program_id(0)
  r = pl.program_id(1)
  inv = 1.0 / jnp.maximum(cnt[:, 0:1], 1.0)
  accb = jnp.concatenate([a0[...], a1[...], a2[...], a3[...]], axis=1)
  contrib = jnp.dot(accb * inv, w[0], preferred_element_type=jnp.float32)

  @pl.when(r == 0)
  def _():
    hb = jnp.concatenate([h0[...], h1[...], h2[...], h3[...]], axis=1)
    acc_s[...] = jnp.dot(hb, root[...], preferred_element_type=jnp.float32) + b[...] + contrib

  @pl.when(r != 0)
  def _():
    acc_s[...] = acc_s[...] + contrib

  @pl.when(r == R - 1)
  def _():
    res = acc_s[...]
    oh = (lax.broadcasted_iota(jnp.int32, (B, BN), 0) == batch2[0]).astype(jnp.float32)
    rsum = jnp.dot(res, jnp.ones((D, 1), jnp.float32), preferred_element_type=jnp.float32)
    pv = jnp.dot(oh, rsum, preferred_element_type=jnp.float32)
    pc = jnp.dot(oh, jnp.ones((BN, 1), jnp.float32), preferred_element_type=jnp.float32)

    @pl.when(i == 0)
    def _():
      psum_s[...] = pv
      pcnt_s[...] = pc

    @pl.when(i != 0)
    def _():
      psum_s[...] = psum_s[...] + pv
      pcnt_s[...] = pcnt_s[...] + pc

    @pl.when(i == NB - 1)
    def _():
      final[...] = psum_s[...] / (jnp.float32(D) * jnp.maximum(pcnt_s[...], 1.0))


def _tc_layer_call(h_slabs, acc_slabs, cnt, w, root, b, last, batch2=None):
  f32 = jnp.float32
  h_spec = [pl.BlockSpec((BN, SLAB), lambda i, r: (i, 0)) for _ in range(NQ)]
  a_spec = [pl.BlockSpec((BN, SLAB), lambda i, r: (r * NB + i, 0)) for _ in range(NQ)]
  cnt_spec = pl.BlockSpec((BN, 16), lambda i, r: (r * NB + i, 0))
  w_spec = pl.BlockSpec((1, D, D), lambda i, r: (r, 0, 0))
  root_spec = pl.BlockSpec((D, D), lambda i, r: (0, 0))
  b_spec = pl.BlockSpec((1, D), lambda i, r: (0, 0))
  params = pltpu.CompilerParams(dimension_semantics=("arbitrary", "arbitrary"))
  if not last:
    return pl.pallas_call(
        _tc_layer0,
        grid=(NB, R),
        in_specs=h_spec + a_spec + [cnt_spec, w_spec, root_spec, b_spec],
        out_specs=[pl.BlockSpec((BN, SLAB), lambda i, r: (i, 0)) for _ in range(NQ)],
        out_shape=[jax.ShapeDtypeStruct((N, SLAB), f32) for _ in range(NQ)],
        scratch_shapes=[pltpu.VMEM((BN, D), f32)],
        compiler_params=params,
    )(*h_slabs, *acc_slabs, cnt, w, root, b)
  batch_spec = pl.BlockSpec((1, 1, BN), lambda i, r: (i, 0, 0))
  return pl.pallas_call(
      _tc_layer1,
      grid=(NB, R),
      in_specs=h_spec + a_spec + [cnt_spec, w_spec, root_spec, b_spec, batch_spec],
      out_specs=pl.BlockSpec((B, 1), lambda i, r: (0, 0)),
      out_shape=jax.ShapeDtypeStruct((B, 1), f32),
      scratch_shapes=[pltpu.VMEM((BN, D), f32), pltpu.VMEM((B, 1), f32),
                      pltpu.VMEM((B, 1), f32)],
      compiler_params=params,
  )(*h_slabs, *acc_slabs, cnt, w, root, b, batch2)


def kernel(atom_type, edge_index, edge_type, batch, emb, W0, root0, b0, W1, root1, b1):
  i32 = jnp.int32
  src = edge_index[0].astype(i32)
  dst = edge_index[1].astype(i32)
  et = edge_type.astype(i32)

  pad_e = E_PAD - E
  ar = jnp.arange(pad_e, dtype=i32)
  src2 = jnp.concatenate([src, ar % N]).reshape(NS, CPT, CHUNK)
  dstadj2 = jnp.concatenate(
      [et * N + dst, R * N + (ar % (ACC_R - R * N))]).reshape(NS, CPT, CHUNK)
  atom2 = jnp.concatenate(
      [atom_type.astype(i32), jnp.arange(NH - N, dtype=i32) % VOCAB]).reshape(4, 20, CHUNK)
  batch2 = batch.astype(i32).reshape(NB, 1, BN)

  e_slabs = tuple(emb.reshape(VOCAB, NQ, SLAB)[:, q, :] for q in range(NQ))

  *h0_slabs, cnt = _sc_prep_call(e_slabs, atom2, dstadj2)
  acc0 = _sc_edge_call(tuple(h0_slabs), src2, dstadj2)
  hm = _tc_layer_call(tuple(h0_slabs), tuple(acc0), cnt, W0, root0,
                      b0.reshape(1, D), last=False)
  acc1 = _sc_edge_call(tuple(hm), src2, dstadj2)
  final = _tc_layer_call(tuple(hm), tuple(acc1), cnt, W1, root1,
                         b1.reshape(1, D), last=True, batch2=batch2)
  return final[:, 0]


# speedup vs baseline: 13.9787x; 1.2693x over previous
import functools

import jax
import jax.numpy as jnp
from jax import lax
from jax.experimental import pallas as pl
from jax.experimental.pallas import tpu as pltpu
from jax.experimental.pallas import tpu_sc as plsc

N = 10000
E = 320000
D = 128
R = 4
VOCAB = 100
B = 256

NS = 16
SLAB = 32
NQ = 4
CHUNK = 128
ECH = 2512
E_PAD = ECH * CHUNK
CPT = ECH // NS
ACC_R = 40960
RPT = ACC_R // NS
NH = 10240
BN = 2000
NB = N // BN

_mesh = lambda: plsc.VectorSubcoreMesh(core_axis_name="c", subcore_axis_name="s")


def _sc_prep(e0, e1, e2, e3, atom2, dstadj2, ones_in, zcnt,
             h0, h1, h2, h3, cnt_out,
             cnt_sh, aidx, didx, rows, ones_v, sem):
  c = lax.axis_index("c")
  s = lax.axis_index("s")
  e_refs = (e0, e1, e2, e3)
  h_refs = (h0, h1, h2, h3)

  @pl.when(c == 1)
  def _():
    part = s % 4
    pltpu.sync_copy(atom2.at[part], aidx)
    for q in range(NQ):
      @pl.when(s // 4 == q)
      def _(q=q):
        def chunk(k, carry):
          base = part * 2560 + k * CHUNK
          pltpu.async_copy(e_refs[q].at[aidx.at[k]], rows, sem).wait()
          pltpu.sync_copy(rows, h_refs[q].at[pl.ds(base, CHUNK)])
          return carry
        lax.fori_loop(0, 20, chunk, 0)

  @pl.when(c == 0)
  def _():
    pltpu.sync_copy(ones_in, ones_v)
    pltpu.sync_copy(zcnt.at[pl.ds(s * RPT, RPT)], cnt_sh.at[pl.ds(s * RPT, RPT)])
    pltpu.sync_copy(dstadj2.at[s], didx)
    plsc.subcore_barrier()

    def chunk(k, carry):
      pltpu.sync_copy(ones_v, cnt_sh.at[didx.at[k]], add=True)
      return carry
    lax.fori_loop(0, CPT, chunk, 0)
    plsc.subcore_barrier()
    pltpu.sync_copy(cnt_sh.at[pl.ds(s * RPT, RPT)], cnt_out.at[pl.ds(s * RPT, RPT)])


def _sc_edge(h0, h1, h2, h3, src2, dstadj2, zacc,
             a0, a1, a2, a3,
             acc_sh, sidx, didx, rows, gsems, ssems):
  c = lax.axis_index("c")
  s = lax.axis_index("s")
  h_refs = (h0, h1, h2, h3)
  a_refs = (a0, a1, a2, a3)

  segs = ((0, 80, 20, False), (80, 77, 19, True))

  for p in (0, 1):
    for cc in (0, 1):
      @pl.when(c == cc)
      def _(q=2 * cc + p):
        h = h_refs[q]
        aout = a_refs[q]
        pltpu.sync_copy(zacc.at[pl.ds(s * RPT, RPT)], acc_sh.at[pl.ds(s * RPT, RPT)])
        plsc.subcore_barrier()

        for row_lo, n_rows, n_quads, tail in segs:
          pltpu.sync_copy(src2.at[s].at[pl.ds(row_lo, n_rows)],
                          sidx.at[pl.ds(0, n_rows)])
          pltpu.sync_copy(dstadj2.at[s].at[pl.ds(row_lo, n_rows)],
                          didx.at[pl.ds(0, n_rows)])

          def quad(j, carry):
            k0 = 4 * j
            gs = [pltpu.async_copy(h.at[sidx.at[k0 + u]], rows[u], gsems[u])
                  for u in range(4)]
            ss = []
            for u in range(4):
              gs[u].wait()
              ss.append(pltpu.async_copy(rows[u], acc_sh.at[didx.at[k0 + u]],
                                         ssems[u], add=True))
            for u in range(4):
              ss[u].wait()
            return carry
          lax.fori_loop(0, n_quads, quad, 0)

          if tail:
            k = n_rows - 1
            pltpu.async_copy(h.at[sidx.at[k]], rows[0], gsems[0]).wait()
            pltpu.async_copy(rows[0], acc_sh.at[didx.at[k]], ssems[0],
                             add=True).wait()
        plsc.subcore_barrier()
        pltpu.sync_copy(acc_sh.at[pl.ds(s * RPT, RPT)], aout.at[pl.ds(s * RPT, RPT)])


def _sc_prep_call(e_slabs, atom2, dstadj2):
  f32 = jnp.float32
  ones_in = jnp.ones((CHUNK, 16), f32)
  zcnt = jnp.zeros((ACC_R, 16), f32)
  fn = pl.kernel(
      _sc_prep,
      out_type=[jax.ShapeDtypeStruct((NH, SLAB), f32) for _ in range(NQ)]
      + [jax.ShapeDtypeStruct((ACC_R, 16), f32)],
      mesh=_mesh(),
      compiler_params=pltpu.CompilerParams(use_tc_tiling_on_sc=False),
      scratch_types=[
          pltpu.VMEM_SHARED((ACC_R, 16), f32),
          pltpu.VMEM((20, CHUNK), jnp.int32),
          pltpu.VMEM((CPT, CHUNK), jnp.int32),
          pltpu.VMEM((CHUNK, SLAB), f32),
          pltpu.VMEM((CHUNK, 16), f32),
          pltpu.SemaphoreType.DMA,
      ],
  )
  return fn(*e_slabs, atom2, dstadj2, ones_in, zcnt)


def _sc_edge_call(h_slabs, src2, dstadj2):
  f32 = jnp.float32
  zacc = jnp.zeros((ACC_R, SLAB), f32)
  fn = pl.kernel(
      _sc_edge,
      out_type=[jax.ShapeDtypeStruct((ACC_R, SLAB), f32) for _ in range(NQ)],
      mesh=_mesh(),
      compiler_params=pltpu.CompilerParams(use_tc_tiling_on_sc=False),
      scratch_types=[
          pltpu.VMEM_SHARED((ACC_R, SLAB), f32),
          pltpu.VMEM((80, CHUNK), jnp.int32),
          pltpu.VMEM((80, CHUNK), jnp.int32),
          [pltpu.VMEM((CHUNK, SLAB), f32) for _ in range(4)],
          [pltpu.SemaphoreType.DMA for _ in range(4)],
          [pltpu.SemaphoreType.DMA for _ in range(4)],
      ],
  )
  return fn(*h_slabs, src2, dstadj2, zacc)


def _tc_layer0(h0, h1, h2, h3, a0, a1, a2, a3, cnt, w, root, b,
               o0, o1, o2, o3, acc_s):
  r = pl.program_id(1)
  inv = 1.0 / jnp.maximum(cnt[:, 0:1], 1.0)
  accb = jnp.concatenate([a0[...], a1[...], a2[...], a3[...]], axis=1)
  contrib = jnp.dot(accb * inv, w[0], preferred_element_type=jnp.float32)

  @pl.when(r == 0)
  def _():
    hb = jnp.concatenate([h0[...], h1[...], h2[...], h3[...]], axis=1)
    acc_s[...] = jnp.dot(hb, root[...], preferred_element_type=jnp.float32) + b[...] + contrib

  @pl.when(r != 0)
  def _():
    acc_s[...] = acc_s[...] + contrib

  @pl.when(r == R - 1)
  def _():
    res = jnp.tanh(acc_s[...])
    outs = (o0, o1, o2, o3)
    for q in range(NQ):
      outs[q][...] = res[:, q * SLAB:(q + 1) * SLAB]


def _tc_layer1(h0, h1, h2, h3, a0, a1, a2, a3, cnt, w, root, b, batch2,
               final, acc_s, psum_s, pcnt_s):
  i = pl.program_id(0)
  r = pl.program_id(1)
  inv = 1.0 / jnp.maximum(cnt[:, 0:1], 1.0)
  accb = jnp.concatenate([a0[...], a1[...], a2[...], a3[...]], axis=1)
  contrib = jnp.dot(accb * inv, w[0], preferred_element_type=jnp.float32)

  @pl.when(r == 0)
  def _():
    hb = jnp.concatenate([h0[...], h1[...], h2[...], h3[...]], axis=1)
    acc_s[...] = jnp.dot(hb, root[...], preferred_element_type=jnp.float32) + b[...] + contrib

  @pl.when(r != 0)
  def _():
    acc_s[...] = acc_s[...] + contrib

  @pl.when(r == R - 1)
  def _():
    res = acc_s[...]
    oh = (lax.broadcasted_iota(jnp.int32, (B, BN), 0) == batch2[0]).astype(jnp.float32)
    rsum = jnp.dot(res, jnp.ones((D, 1), jnp.float32), preferred_element_type=jnp.float32)
    pv = jnp.dot(oh, rsum, preferred_element_type=jnp.float32)
    pc = jnp.dot(oh, jnp.ones((BN, 1), jnp.float32), preferred_element_type=jnp.float32)

    @pl.when(i == 0)
    def _():
      psum_s[...] = pv
      pcnt_s[...] = pc

    @pl.when(i != 0)
    def _():
      psum_s[...] = psum_s[...] + pv
      pcnt_s[...] = pcnt_s[...] + pc

    @pl.when(i == NB - 1)
    def _():
      final[...] = psum_s[...] / (jnp.float32(D) * jnp.maximum(pcnt_s[...], 1.0))


def _tc_layer_call(h_slabs, acc_slabs, cnt, w, root, b, last, batch2=None):
  f32 = jnp.float32
  h_spec = [pl.BlockSpec((BN, SLAB), lambda i, r: (i, 0)) for _ in range(NQ)]
  a_spec = [pl.BlockSpec((BN, SLAB), lambda i, r: (r * NB + i, 0)) for _ in range(NQ)]
  cnt_spec = pl.BlockSpec((BN, 16), lambda i, r: (r * NB + i, 0))
  w_spec = pl.BlockSpec((1, D, D), lambda i, r: (r, 0, 0))
  root_spec = pl.BlockSpec((D, D), lambda i, r: (0, 0))
  b_spec = pl.BlockSpec((1, D), lambda i, r: (0, 0))
  params = pltpu.CompilerParams(dimension_semantics=("arbitrary", "arbitrary"))
  if not last:
    return pl.pallas_call(
        _tc_layer0,
        grid=(NB, R),
        in_specs=h_spec + a_spec + [cnt_spec, w_spec, root_spec, b_spec],
        out_specs=[pl.BlockSpec((BN, SLAB), lambda i, r: (i, 0)) for _ in range(NQ)],
        out_shape=[jax.ShapeDtypeStruct((N, SLAB), f32) for _ in range(NQ)],
        scratch_shapes=[pltpu.VMEM((BN, D), f32)],
        compiler_params=params,
    )(*h_slabs, *acc_slabs, cnt, w, root, b)
  batch_spec = pl.BlockSpec((1, 1, BN), lambda i, r: (i, 0, 0))
  return pl.pallas_call(
      _tc_layer1,
      grid=(NB, R),
      in_specs=h_spec + a_spec + [cnt_spec, w_spec, root_spec, b_spec, batch_spec],
      out_specs=pl.BlockSpec((B, 1), lambda i, r: (0, 0)),
      out_shape=jax.ShapeDtypeStruct((B, 1), f32),
      scratch_shapes=[pltpu.VMEM((BN, D), f32), pltpu.VMEM((B, 1), f32),
                      pltpu.VMEM((B, 1), f32)],
      compiler_params=params,
  )(*h_slabs, *acc_slabs, cnt, w, root, b, batch2)


def kernel(atom_type, edge_index, edge_type, batch, emb, W0, root0, b0, W1, root1, b1):
  i32 = jnp.int32
  src = edge_index[0].astype(i32)
  dst = edge_index[1].astype(i32)
  et = edge_type.astype(i32)

  pad_e = E_PAD - E
  ar = jnp.arange(pad_e, dtype=i32)
  src2 = jnp.concatenate([src, ar % N]).reshape(NS, CPT, CHUNK)
  dstadj2 = jnp.concatenate(
      [et * N + dst, R * N + (ar % (ACC_R - R * N))]).reshape(NS, CPT, CHUNK)
  atom2 = jnp.concatenate(
      [atom_type.astype(i32), jnp.arange(NH - N, dtype=i32) % VOCAB]).reshape(4, 20, CHUNK)
  batch2 = batch.astype(i32).reshape(NB, 1, BN)

  e_slabs = tuple(emb.reshape(VOCAB, NQ, SLAB)[:, q, :] for q in range(NQ))

  *h0_slabs, cnt = _sc_prep_call(e_slabs, atom2, dstadj2)
  acc0 = _sc_edge_call(tuple(h0_slabs), src2, dstadj2)
  hm = _tc_layer_call(tuple(h0_slabs), tuple(acc0), cnt, W0, root0,
                      b0.reshape(1, D), last=False)
  acc1 = _sc_edge_call(tuple(hm), src2, dstadj2)
  final = _tc_layer_call(tuple(hm), tuple(acc1), cnt, W1, root1,
                         b1.reshape(1, D), last=True, batch2=batch2)
  return final[:, 0]


# speedup vs baseline: 14.2862x; 1.0220x over previous
import functools

import jax
import jax.numpy as jnp
from jax import lax
from jax.experimental import pallas as pl
from jax.experimental.pallas import tpu as pltpu
from jax.experimental.pallas import tpu_sc as plsc

N = 10000
E = 320000
D = 128
R = 4
VOCAB = 100
B = 256

NS = 16
SLAB = 32
NQ = 4
CHUNK = 128
ECH = 2512
E_PAD = ECH * CHUNK
CPT = ECH // NS
ACC_R = 40960
RPT = ACC_R // NS
NH = 10240
BN = 1000
NB = N // BN

_mesh = lambda: plsc.VectorSubcoreMesh(core_axis_name="c", subcore_axis_name="s")


def _sc_prep(e0, e1, e2, e3, atom2, dstadj2, ones_in, zcnt,
             h0, h1, h2, h3, cnt_out,
             cnt_sh, aidx, didx, rows, ones_v, sem):
  c = lax.axis_index("c")
  s = lax.axis_index("s")
  e_refs = (e0, e1, e2, e3)
  h_refs = (h0, h1, h2, h3)

  @pl.when(c == 1)
  def _():
    part = s % 4
    pltpu.sync_copy(atom2.at[part], aidx)
    for q in range(NQ):
      @pl.when(s // 4 == q)
      def _(q=q):
        def chunk(k, carry):
          base = part * 2560 + k * CHUNK
          pltpu.async_copy(e_refs[q].at[aidx.at[k]], rows, sem).wait()
          pltpu.sync_copy(rows, h_refs[q].at[pl.ds(base, CHUNK)])
          return carry
        lax.fori_loop(0, 20, chunk, 0)

  @pl.when(c == 0)
  def _():
    pltpu.sync_copy(ones_in, ones_v)
    pltpu.sync_copy(zcnt.at[pl.ds(s * RPT, RPT)], cnt_sh.at[pl.ds(s * RPT, RPT)])
    pltpu.sync_copy(dstadj2.at[s], didx)
    plsc.subcore_barrier()

    def chunk(k, carry):
      pltpu.sync_copy(ones_v, cnt_sh.at[didx.at[k]], add=True)
      return carry
    lax.fori_loop(0, CPT, chunk, 0)
    plsc.subcore_barrier()
    pltpu.sync_copy(cnt_sh.at[pl.ds(s * RPT, RPT)], cnt_out.at[pl.ds(s * RPT, RPT)])


def _sc_edge(h0, h1, h2, h3, src2, dstadj2, zacc,
             a0, a1, a2, a3,
             acc_sh, sidx, didx, rows, gsems, ssems):
  c = lax.axis_index("c")
  s = lax.axis_index("s")
  h_refs = (h0, h1, h2, h3)
  a_refs = (a0, a1, a2, a3)

  segs = ((0, 80, 20, False), (80, 77, 19, True))

  for p in (0, 1):
    for cc in (0, 1):
      @pl.when(c == cc)
      def _(q=2 * cc + p):
        h = h_refs[q]
        aout = a_refs[q]
        pltpu.sync_copy(zacc.at[pl.ds(s * RPT, RPT)], acc_sh.at[pl.ds(s * RPT, RPT)])
        plsc.subcore_barrier()

        for row_lo, n_rows, n_quads, tail in segs:
          pltpu.sync_copy(src2.at[s].at[pl.ds(row_lo, n_rows)],
                          sidx.at[pl.ds(0, n_rows)])
          pltpu.sync_copy(dstadj2.at[s].at[pl.ds(row_lo, n_rows)],
                          didx.at[pl.ds(0, n_rows)])

          def quad(j, carry):
            k0 = 4 * j
            gs = [pltpu.async_copy(h.at[sidx.at[k0 + u]], rows[u], gsems[u])
                  for u in range(4)]
            ss = []
            for u in range(4):
              gs[u].wait()
              ss.append(pltpu.async_copy(rows[u], acc_sh.at[didx.at[k0 + u]],
                                         ssems[u], add=True))
            for u in range(4):
              ss[u].wait()
            return carry
          lax.fori_loop(0, n_quads, quad, 0)

          if tail:
            k = n_rows - 1
            pltpu.async_copy(h.at[sidx.at[k]], rows[0], gsems[0]).wait()
            pltpu.async_copy(rows[0], acc_sh.at[didx.at[k]], ssems[0],
                             add=True).wait()
        plsc.subcore_barrier()
        pltpu.sync_copy(acc_sh.at[pl.ds(s * RPT, RPT)], aout.at[pl.ds(s * RPT, RPT)])


def _sc_prep_call(e_slabs, atom2, dstadj2):
  f32 = jnp.float32
  ones_in = jnp.ones((CHUNK, 16), f32)
  zcnt = jnp.zeros((ACC_R, 16), f32)
  fn = pl.kernel(
      _sc_prep,
      out_type=[jax.ShapeDtypeStruct((NH, SLAB), f32) for _ in range(NQ)]
      + [jax.ShapeDtypeStruct((ACC_R, 16), f32)],
      mesh=_mesh(),
      compiler_params=pltpu.CompilerParams(use_tc_tiling_on_sc=False),
      scratch_types=[
          pltpu.VMEM_SHARED((ACC_R, 16), f32),
          pltpu.VMEM((20, CHUNK), jnp.int32),
          pltpu.VMEM((CPT, CHUNK), jnp.int32),
          pltpu.VMEM((CHUNK, SLAB), f32),
          pltpu.VMEM((CHUNK, 16), f32),
          pltpu.SemaphoreType.DMA,
      ],
  )
  return fn(*e_slabs, atom2, dstadj2, ones_in, zcnt)


def _sc_edge_call(h_slabs, src2, dstadj2):
  f32 = jnp.float32
  zacc = jnp.zeros((ACC_R, SLAB), f32)
  fn = pl.kernel(
      _sc_edge,
      out_type=[jax.ShapeDtypeStruct((ACC_R, SLAB), f32) for _ in range(NQ)],
      mesh=_mesh(),
      compiler_params=pltpu.CompilerParams(use_tc_tiling_on_sc=False),
      scratch_types=[
          pltpu.VMEM_SHARED((ACC_R, SLAB), f32),
          pltpu.VMEM((80, CHUNK), jnp.int32),
          pltpu.VMEM((80, CHUNK), jnp.int32),
          [pltpu.VMEM((CHUNK, SLAB), f32) for _ in range(4)],
          [pltpu.SemaphoreType.DMA for _ in range(4)],
          [pltpu.SemaphoreType.DMA for _ in range(4)],
      ],
  )
  return fn(*h_slabs, src2, dstadj2, zacc)


def _tc_layer0(*refs):
  h = refs[0:NQ]
  accs = refs[NQ:NQ + NQ * R]
  cnts = refs[NQ + NQ * R:NQ + NQ * R + R]
  w, root, b = refs[NQ + NQ * R + R:NQ + NQ * R + R + 3]
  outs = refs[NQ + NQ * R + R + 3:]

  hb = jnp.concatenate([h[q][...] for q in range(NQ)], axis=1)
  out = jnp.dot(hb, root[...], preferred_element_type=jnp.float32) + b[...]
  for r in range(R):
    inv = 1.0 / jnp.maximum(cnts[r][:, 0:1], 1.0)
    accb = jnp.concatenate([accs[q * R + r][...] for q in range(NQ)], axis=1)
    out = out + jnp.dot(accb * inv, w[r], preferred_element_type=jnp.float32)
  res = jnp.tanh(out)
  for q in range(NQ):
    outs[q][...] = res[:, q * SLAB:(q + 1) * SLAB]


def _tc_layer1(*refs):
  h = refs[0:NQ]
  accs = refs[NQ:NQ + NQ * R]
  cnts = refs[NQ + NQ * R:NQ + NQ * R + R]
  w, root, b, batch2 = refs[NQ + NQ * R + R:NQ + NQ * R + R + 4]
  final, psum_s, pcnt_s = refs[NQ + NQ * R + R + 4:]
  i = pl.program_id(0)

  hb = jnp.concatenate([h[q][...] for q in range(NQ)], axis=1)
  out = jnp.dot(hb, root[...], preferred_element_type=jnp.float32) + b[...]
  for r in range(R):
    inv = 1.0 / jnp.maximum(cnts[r][:, 0:1], 1.0)
    accb = jnp.concatenate([accs[q * R + r][...] for q in range(NQ)], axis=1)
    out = out + jnp.dot(accb * inv, w[r], preferred_element_type=jnp.float32)

  oh = (lax.broadcasted_iota(jnp.int32, (B, BN), 0) == batch2[0]).astype(jnp.float32)
  rsum = jnp.dot(out, jnp.ones((D, 1), jnp.float32), preferred_element_type=jnp.float32)
  pv = jnp.dot(oh, rsum, preferred_element_type=jnp.float32)
  pc = jnp.dot(oh, jnp.ones((BN, 1), jnp.float32), preferred_element_type=jnp.float32)

  @pl.when(i == 0)
  def _():
    psum_s[...] = pv
    pcnt_s[...] = pc

  @pl.when(i != 0)
  def _():
    psum_s[...] = psum_s[...] + pv
    pcnt_s[...] = pcnt_s[...] + pc

  @pl.when(i == NB - 1)
  def _():
    final[...] = psum_s[...] / (jnp.float32(D) * jnp.maximum(pcnt_s[...], 1.0))


def _tc_layer_call(h_slabs, acc_slabs, cnt, w, root, b, last, batch2=None):
  f32 = jnp.float32
  h_spec = [pl.BlockSpec((BN, SLAB), lambda i: (i, 0)) for _ in range(NQ)]
  a_spec = [pl.BlockSpec((BN, SLAB), lambda i, r=r: (r * NB + i, 0))
            for _ in range(NQ) for r in range(R)]
  a_args = [acc_slabs[q] for q in range(NQ) for _ in range(R)]
  cnt_spec = [pl.BlockSpec((BN, 16), lambda i, r=r: (r * NB + i, 0))
              for r in range(R)]
  w_spec = pl.BlockSpec((R, D, D), lambda i: (0, 0, 0))
  root_spec = pl.BlockSpec((D, D), lambda i: (0, 0))
  b_spec = pl.BlockSpec((1, D), lambda i: (0, 0))
  params = pltpu.CompilerParams(dimension_semantics=("arbitrary",))
  if not last:
    return pl.pallas_call(
        _tc_layer0,
        grid=(NB,),
        in_specs=h_spec + a_spec + cnt_spec + [w_spec, root_spec, b_spec],
        out_specs=[pl.BlockSpec((BN, SLAB), lambda i: (i, 0)) for _ in range(NQ)],
        out_shape=[jax.ShapeDtypeStruct((N, SLAB), f32) for _ in range(NQ)],
        compiler_params=params,
    )(*h_slabs, *a_args, *([cnt] * R), w, root, b)
  batch_spec = pl.BlockSpec((1, 1, BN), lambda i: (i, 0, 0))
  return pl.pallas_call(
      _tc_layer1,
      grid=(NB,),
      in_specs=h_spec + a_spec + cnt_spec + [w_spec, root_spec, b_spec, batch_spec],
      out_specs=pl.BlockSpec((B, 1), lambda i: (0, 0)),
      out_shape=jax.ShapeDtypeStruct((B, 1), f32),
      scratch_shapes=[pltpu.VMEM((B, 1), f32), pltpu.VMEM((B, 1), f32)],
      compiler_params=params,
  )(*h_slabs, *a_args, *([cnt] * R), w, root, b, batch2)


def kernel(atom_type, edge_index, edge_type, batch, emb, W0, root0, b0, W1, root1, b1):
  i32 = jnp.int32
  src = edge_index[0].astype(i32)
  dst = edge_index[1].astype(i32)
  et = edge_type.astype(i32)

  pad_e = E_PAD - E
  ar = jnp.arange(pad_e, dtype=i32)
  src2 = jnp.concatenate([src, ar % N]).reshape(NS, CPT, CHUNK)
  dstadj2 = jnp.concatenate(
      [et * N + dst, R * N + (ar % (ACC_R - R * N))]).reshape(NS, CPT, CHUNK)
  atom2 = jnp.concatenate(
      [atom_type.astype(i32), jnp.arange(NH - N, dtype=i32) % VOCAB]).reshape(4, 20, CHUNK)
  batch2 = batch.astype(i32).reshape(NB, 1, BN)

  e_slabs = tuple(emb.reshape(VOCAB, NQ, SLAB)[:, q, :] for q in range(NQ))

  *h0_slabs, cnt = _sc_prep_call(e_slabs, atom2, dstadj2)
  acc0 = _sc_edge_call(tuple(h0_slabs), src2, dstadj2)
  hm = _tc_layer_call(tuple(h0_slabs), tuple(acc0), cnt, W0, root0,
                      b0.reshape(1, D), last=False)
  acc1 = _sc_edge_call(tuple(hm), src2, dstadj2)
  final = _tc_layer_call(tuple(hm), tuple(acc1), cnt, W1, root1,
                         b1.reshape(1, D), last=True, batch2=batch2)
  return final[:, 0]


# speedup vs baseline: 14.6534x; 1.0257x over previous
import functools

import jax
import jax.numpy as jnp
from jax import lax
from jax.experimental import pallas as pl
from jax.experimental.pallas import tpu as pltpu
from jax.experimental.pallas import tpu_sc as plsc

N = 10000
E = 320000
D = 128
R = 4
VOCAB = 100
B = 256

NS = 16
SLAB = 32
NQ = 4
CHUNK = 128
ECH = 2560
E_PAD = ECH * CHUNK
CPT = ECH // NS
SEG = 32
NBUF = 8
ACC_R = 40960
RPT = ACC_R // NS
NH = 10240
BN = 1000
NB = N // BN

_mesh = lambda: plsc.VectorSubcoreMesh(core_axis_name="c", subcore_axis_name="s")


def _sc_prep(e0, e1, e2, e3, atom2, dstadj2, ones_in, zcnt,
             h0, h1, h2, h3, cnt_out,
             cnt_sh, aidx, didx, rows, ones_v, sem):
  c = lax.axis_index("c")
  s = lax.axis_index("s")
  e_refs = (e0, e1, e2, e3)
  h_refs = (h0, h1, h2, h3)

  @pl.when(c == 1)
  def _():
    part = s % 4
    pltpu.sync_copy(atom2.at[part], aidx)
    for q in range(NQ):
      @pl.when(s // 4 == q)
      def _(q=q):
        def chunk(k, carry):
          base = part * 2560 + k * CHUNK
          pltpu.async_copy(e_refs[q].at[aidx.at[k]], rows, sem).wait()
          pltpu.sync_copy(rows, h_refs[q].at[pl.ds(base, CHUNK)])
          return carry
        lax.fori_loop(0, 20, chunk, 0)

  @pl.when(c == 0)
  def _():
    pltpu.sync_copy(ones_in, ones_v)
    pltpu.sync_copy(zcnt.at[pl.ds(s * RPT, RPT)], cnt_sh.at[pl.ds(s * RPT, RPT)])
    pltpu.sync_copy(dstadj2.at[s], didx)
    plsc.subcore_barrier()

    def chunk(k, carry):
      pltpu.sync_copy(ones_v, cnt_sh.at[didx.at[k]], add=True)
      return carry
    lax.fori_loop(0, CPT, chunk, 0)
    plsc.subcore_barrier()
    pltpu.sync_copy(cnt_sh.at[pl.ds(s * RPT, RPT)], cnt_out.at[pl.ds(s * RPT, RPT)])


def _sc_edge(h0, h1, h2, h3, src2, dstadj2, zacc,
             a0, a1, a2, a3,
             acc_sh, sidx, didx, rows, gsems, ssems):
  c = lax.axis_index("c")
  s = lax.axis_index("s")
  h_refs = (h0, h1, h2, h3)
  a_refs = (a0, a1, a2, a3)

  for p in (0, 1):
    for cc in (0, 1):
      @pl.when(c == cc)
      def _(q=2 * cc + p):
        h = h_refs[q]
        aout = a_refs[q]
        pltpu.sync_copy(zacc.at[pl.ds(s * RPT, RPT)], acc_sh.at[pl.ds(s * RPT, RPT)])
        plsc.subcore_barrier()

        for seg in range(CPT // SEG):
          pltpu.sync_copy(src2.at[s].at[pl.ds(seg * SEG, SEG)], sidx)
          pltpu.sync_copy(dstadj2.at[s].at[pl.ds(seg * SEG, SEG)], didx)

          def octet(j, carry):
            k0 = NBUF * j
            gs = [pltpu.async_copy(h.at[sidx.at[k0 + u]], rows[u], gsems[u])
                  for u in range(NBUF)]
            ss = []
            for u in range(NBUF):
              gs[u].wait()
              ss.append(pltpu.async_copy(rows[u], acc_sh.at[didx.at[k0 + u]],
                                         ssems[u], add=True))
            for u in range(NBUF):
              ss[u].wait()
            return carry
          lax.fori_loop(0, SEG // NBUF, octet, 0)
        plsc.subcore_barrier()
        pltpu.sync_copy(acc_sh.at[pl.ds(s * RPT, RPT)], aout.at[pl.ds(s * RPT, RPT)])


def _sc_prep_call(e_slabs, atom2, dstadj2):
  f32 = jnp.float32
  ones_in = jnp.ones((CHUNK, 16), f32)
  zcnt = jnp.zeros((ACC_R, 16), f32)
  fn = pl.kernel(
      _sc_prep,
      out_type=[jax.ShapeDtypeStruct((NH, SLAB), f32) for _ in range(NQ)]
      + [jax.ShapeDtypeStruct((ACC_R, 16), f32)],
      mesh=_mesh(),
      compiler_params=pltpu.CompilerParams(use_tc_tiling_on_sc=False),
      scratch_types=[
          pltpu.VMEM_SHARED((ACC_R, 16), f32),
          pltpu.VMEM((20, CHUNK), jnp.int32),
          pltpu.VMEM((CPT, CHUNK), jnp.int32),
          pltpu.VMEM((CHUNK, SLAB), f32),
          pltpu.VMEM((CHUNK, 16), f32),
          pltpu.SemaphoreType.DMA,
      ],
  )
  return fn(*e_slabs, atom2, dstadj2, ones_in, zcnt)


def _sc_edge_call(h_slabs, src2, dstadj2):
  f32 = jnp.float32
  zacc = jnp.zeros((ACC_R, SLAB), f32)
  fn = pl.kernel(
      _sc_edge,
      out_type=[jax.ShapeDtypeStruct((ACC_R, SLAB), f32) for _ in range(NQ)],
      mesh=_mesh(),
      compiler_params=pltpu.CompilerParams(use_tc_tiling_on_sc=False),
      scratch_types=[
          pltpu.VMEM_SHARED((ACC_R, SLAB), f32),
          pltpu.VMEM((SEG, CHUNK), jnp.int32),
          pltpu.VMEM((SEG, CHUNK), jnp.int32),
          [pltpu.VMEM((CHUNK, SLAB), f32) for _ in range(NBUF)],
          [pltpu.SemaphoreType.DMA for _ in range(NBUF)],
          [pltpu.SemaphoreType.DMA for _ in range(NBUF)],
      ],
  )
  return fn(*h_slabs, src2, dstadj2, zacc)


def _tc_layer0(*refs):
  h = refs[0:NQ]
  accs = refs[NQ:NQ + NQ * R]
  cnts = refs[NQ + NQ * R:NQ + NQ * R + R]
  w, root, b = refs[NQ + NQ * R + R:NQ + NQ * R + R + 3]
  outs = refs[NQ + NQ * R + R + 3:]

  hb = jnp.concatenate([h[q][...] for q in range(NQ)], axis=1)
  out = jnp.dot(hb, root[...], preferred_element_type=jnp.float32) + b[...]
  for r in range(R):
    inv = 1.0 / jnp.maximum(cnts[r][:, 0:1], 1.0)
    accb = jnp.concatenate([accs[q * R + r][...] for q in range(NQ)], axis=1)
    out = out + jnp.dot(accb * inv, w[r], preferred_element_type=jnp.float32)
  res = jnp.tanh(out)
  for q in range(NQ):
    outs[q][...] = res[:, q * SLAB:(q + 1) * SLAB]


def _tc_layer1(*refs):
  h = refs[0:NQ]
  accs = refs[NQ:NQ + NQ * R]
  cnts = refs[NQ + NQ * R:NQ + NQ * R + R]
  w, root, b, batch2 = refs[NQ + NQ * R + R:NQ + NQ * R + R + 4]
  final, psum_s, pcnt_s = refs[NQ + NQ * R + R + 4:]
  i = pl.program_id(0)

  hb = jnp.concatenate([h[q][...] for q in range(NQ)], axis=1)
  out = jnp.dot(hb, root[...], preferred_element_type=jnp.float32) + b[...]
  for r in range(R):
    inv = 1.0 / jnp.maximum(cnts[r][:, 0:1], 1.0)
    accb = jnp.concatenate([accs[q * R + r][...] for q in range(NQ)], axis=1)
    out = out + jnp.dot(accb * inv, w[r], preferred_element_type=jnp.float32)

  oh = (lax.broadcasted_iota(jnp.int32, (B, BN), 0) == batch2[0]).astype(jnp.float32)
  rsum = jnp.dot(out, jnp.ones((D, 1), jnp.float32), preferred_element_type=jnp.float32)
  pv = jnp.dot(oh, rsum, preferred_element_type=jnp.float32)
  pc = jnp.dot(oh, jnp.ones((BN, 1), jnp.float32), preferred_element_type=jnp.float32)

  @pl.when(i == 0)
  def _():
    psum_s[...] = pv
    pcnt_s[...] = pc

  @pl.when(i != 0)
  def _():
    psum_s[...] = psum_s[...] + pv
    pcnt_s[...] = pcnt_s[...] + pc

  @pl.when(i == NB - 1)
  def _():
    final[...] = psum_s[...] / (jnp.float32(D) * jnp.maximum(pcnt_s[...], 1.0))


def _tc_layer_call(h_slabs, acc_slabs, cnt, w, root, b, last, batch2=None):
  f32 = jnp.float32
  h_spec = [pl.BlockSpec((BN, SLAB), lambda i: (i, 0)) for _ in range(NQ)]
  a_spec = [pl.BlockSpec((BN, SLAB), lambda i, r=r: (r * NB + i, 0))
            for _ in range(NQ) for r in range(R)]
  a_args = [acc_slabs[q] for q in range(NQ) for _ in range(R)]
  cnt_spec = [pl.BlockSpec((BN, 16), lambda i, r=r: (r * NB + i, 0))
              for r in range(R)]
  w_spec = pl.BlockSpec((R, D, D), lambda i: (0, 0, 0))
  root_spec = pl.BlockSpec((D, D), lambda i: (0, 0))
  b_spec = pl.BlockSpec((1, D), lambda i: (0, 0))
  params = pltpu.CompilerParams(dimension_semantics=("arbitrary",))
  if not last:
    return pl.pallas_call(
        _tc_layer0,
        grid=(NB,),
        in_specs=h_spec + a_spec + cnt_spec + [w_spec, root_spec, b_spec],
        out_specs=[pl.BlockSpec((BN, SLAB), lambda i: (i, 0)) for _ in range(NQ)],
        out_shape=[jax.ShapeDtypeStruct((N, SLAB), f32) for _ in range(NQ)],
        compiler_params=params,
    )(*h_slabs, *a_args, *([cnt] * R), w, root, b)
  batch_spec = pl.BlockSpec((1, 1, BN), lambda i: (i, 0, 0))
  return pl.pallas_call(
      _tc_layer1,
      grid=(NB,),
      in_specs=h_spec + a_spec + cnt_spec + [w_spec, root_spec, b_spec, batch_spec],
      out_specs=pl.BlockSpec((B, 1), lambda i: (0, 0)),
      out_shape=jax.ShapeDtypeStruct((B, 1), f32),
      scratch_shapes=[pltpu.VMEM((B, 1), f32), pltpu.VMEM((B, 1), f32)],
      compiler_params=params,
  )(*h_slabs, *a_args, *([cnt] * R), w, root, b, batch2)


def kernel(atom_type, edge_index, edge_type, batch, emb, W0, root0, b0, W1, root1, b1):
  i32 = jnp.int32
  src = edge_index[0].astype(i32)
  dst = edge_index[1].astype(i32)
  et = edge_type.astype(i32)

  pad_e = E_PAD - E
  ar = jnp.arange(pad_e, dtype=i32)
  src2 = jnp.concatenate([src, ar % N]).reshape(NS, CPT, CHUNK)
  dstadj2 = jnp.concatenate(
      [et * N + dst, R * N + (ar % (ACC_R - R * N))]).reshape(NS, CPT, CHUNK)
  atom2 = jnp.concatenate(
      [atom_type.astype(i32), jnp.arange(NH - N, dtype=i32) % VOCAB]).reshape(4, 20, CHUNK)
  batch2 = batch.astype(i32).reshape(NB, 1, BN)

  e_slabs = tuple(emb.reshape(VOCAB, NQ, SLAB)[:, q, :] for q in range(NQ))

  *h0_slabs, cnt = _sc_prep_call(e_slabs, atom2, dstadj2)
  acc0 = _sc_edge_call(tuple(h0_slabs), src2, dstadj2)
  hm = _tc_layer_call(tuple(h0_slabs), tuple(acc0), cnt, W0, root0,
                      b0.reshape(1, D), last=False)
  acc1 = _sc_edge_call(tuple(hm), src2, dstadj2)
  final = _tc_layer_call(tuple(hm), tuple(acc1), cnt, W1, root1,
                         b1.reshape(1, D), last=True, batch2=batch2)
  return final[:, 0]


# speedup vs baseline: 18.8691x; 1.2877x over previous
import functools

import jax
import jax.numpy as jnp
from jax import lax
from jax.experimental import pallas as pl
from jax.experimental.pallas import tpu as pltpu
from jax.experimental.pallas import tpu_sc as plsc

N = 10000
E = 320000
D = 128
R = 4
VOCAB = 100
B = 256

NS = 16
SLAB = 32
NQ = 4
CHUNK = 128
ECH = 2560
E_PAD = ECH * CHUNK
CPT = ECH // NS
SEG = 32
NBUF = 8
ACC_R = 40960
RPT = ACC_R // NS
NH = 10240
BN = 1000
NB = N // BN

_mesh = lambda: plsc.VectorSubcoreMesh(core_axis_name="c", subcore_axis_name="s")


def _sc_prep(e4, atom4, dstadj2, ones_in, zcnt,
             h0m, cnt_out,
             cnt_sh, aidx, didx, rows, ones_v, sem):
  c = lax.axis_index("c")
  s = lax.axis_index("s")

  @pl.when(c == 1)
  def _():
    part = s % 4
    q = s // 4
    pltpu.sync_copy(atom4.at[s], aidx)

    def chunk(k, carry):
      base = part * 2560 + k * CHUNK
      pltpu.async_copy(e4.at[aidx.at[k]], rows, sem).wait()
      pltpu.sync_copy(rows, h0m.at[pl.ds(base, CHUNK), pl.ds(32 * q, SLAB)])
      return carry
    lax.fori_loop(0, 20, chunk, 0)

  @pl.when(c == 0)
  def _():
    pltpu.sync_copy(ones_in, ones_v)
    pltpu.sync_copy(zcnt.at[pl.ds(s * RPT, RPT)], cnt_sh.at[pl.ds(s * RPT, RPT)])
    pltpu.sync_copy(dstadj2.at[s], didx)
    plsc.subcore_barrier()

    def chunk(k, carry):
      pltpu.sync_copy(ones_v, cnt_sh.at[didx.at[k]], add=True)
      return carry
    lax.fori_loop(0, CPT, chunk, 0)
    plsc.subcore_barrier()
    pltpu.sync_copy(cnt_sh.at[pl.ds(s * RPT, RPT)], cnt_out.at[pl.ds(s * RPT, RPT)])


def _sc_edge(hv, src40, src41, src42, src43, dstadj2, zacc,
             aout,
             acc_sh, sidx, didx, rows, gsems, ssems):
  c = lax.axis_index("c")
  s = lax.axis_index("s")
  src4 = (src40, src41, src42, src43)

  for p in (0, 1):
    for cc in (0, 1):
      @pl.when(c == cc)
      def _(q=2 * cc + p):
        pltpu.sync_copy(zacc.at[pl.ds(s * RPT, RPT)], acc_sh.at[pl.ds(s * RPT, RPT)])
        plsc.subcore_barrier()

        for seg in range(CPT // SEG):
          pltpu.sync_copy(src4[q].at[s].at[pl.ds(seg * SEG, SEG)], sidx)
          pltpu.sync_copy(dstadj2.at[s].at[pl.ds(seg * SEG, SEG)], didx)

          def octet(j, carry):
            k0 = NBUF * j
            gs = [pltpu.async_copy(hv.at[sidx.at[k0 + u]], rows[u], gsems[u])
                  for u in range(NBUF)]
            ss = []
            for u in range(NBUF):
              gs[u].wait()
              ss.append(pltpu.async_copy(rows[u], acc_sh.at[didx.at[k0 + u]],
                                         ssems[u], add=True))
            for u in range(NBUF):
              ss[u].wait()
            return carry
          lax.fori_loop(0, SEG // NBUF, octet, 0)
        plsc.subcore_barrier()
        pltpu.sync_copy(acc_sh.at[pl.ds(s * RPT, RPT)],
                        aout.at[pl.ds(s * RPT, RPT), pl.ds(32 * q, SLAB)])


def _sc_prep_call(e4, atom4, dstadj2):
  f32 = jnp.float32
  ones_in = jnp.ones((CHUNK, 16), f32)
  zcnt = jnp.zeros((ACC_R, 16), f32)
  fn = pl.kernel(
      _sc_prep,
      out_type=[jax.ShapeDtypeStruct((NH, D), f32),
                jax.ShapeDtypeStruct((ACC_R, 16), f32)],
      mesh=_mesh(),
      compiler_params=pltpu.CompilerParams(use_tc_tiling_on_sc=False),
      scratch_types=[
          pltpu.VMEM_SHARED((ACC_R, 16), f32),
          pltpu.VMEM((20, CHUNK), jnp.int32),
          pltpu.VMEM((CPT, CHUNK), jnp.int32),
          pltpu.VMEM((CHUNK, SLAB), f32),
          pltpu.VMEM((CHUNK, 16), f32),
          pltpu.SemaphoreType.DMA,
      ],
  )
  return fn(e4, atom4, dstadj2, ones_in, zcnt)


def _sc_edge_call(hv, src4, dstadj2):
  f32 = jnp.float32
  zacc = jnp.zeros((ACC_R, SLAB), f32)
  fn = pl.kernel(
      _sc_edge,
      out_type=jax.ShapeDtypeStruct((ACC_R, D), f32),
      mesh=_mesh(),
      compiler_params=pltpu.CompilerParams(use_tc_tiling_on_sc=False),
      scratch_types=[
          pltpu.VMEM_SHARED((ACC_R, SLAB), f32),
          pltpu.VMEM((SEG, CHUNK), jnp.int32),
          pltpu.VMEM((SEG, CHUNK), jnp.int32),
          [pltpu.VMEM((CHUNK, SLAB), f32) for _ in range(NBUF)],
          [pltpu.SemaphoreType.DMA for _ in range(NBUF)],
          [pltpu.SemaphoreType.DMA for _ in range(NBUF)],
      ],
  )
  return fn(hv, *src4, dstadj2, zacc)


def _tc_layer0(h, a0, a1, a2, a3, c0, c1, c2, c3, w, root, b, out):
  accs = (a0, a1, a2, a3)
  cnts = (c0, c1, c2, c3)
  res = jnp.dot(h[...], root[...], preferred_element_type=jnp.float32) + b[...]
  for r in range(R):
    inv = 1.0 / jnp.maximum(cnts[r][:, 0:1], 1.0)
    res = res + jnp.dot(accs[r][...] * inv, w[r],
                        preferred_element_type=jnp.float32)
  out[...] = jnp.tanh(res)


def _tc_layer1(h, a0, a1, a2, a3, c0, c1, c2, c3, w, root, b, batch2,
               final, psum_s, pcnt_s):
  accs = (a0, a1, a2, a3)
  cnts = (c0, c1, c2, c3)
  i = pl.program_id(0)
  res = jnp.dot(h[...], root[...], preferred_element_type=jnp.float32) + b[...]
  for r in range(R):
    inv = 1.0 / jnp.maximum(cnts[r][:, 0:1], 1.0)
    res = res + jnp.dot(accs[r][...] * inv, w[r],
                        preferred_element_type=jnp.float32)

  oh = (lax.broadcasted_iota(jnp.int32, (B, BN), 0) == batch2[0]).astype(jnp.float32)
  rsum = jnp.dot(res, jnp.ones((D, 1), jnp.float32), preferred_element_type=jnp.float32)
  pv = jnp.dot(oh, rsum, preferred_element_type=jnp.float32)
  pc = jnp.dot(oh, jnp.ones((BN, 1), jnp.float32), preferred_element_type=jnp.float32)

  @pl.when(i == 0)
  def _():
    psum_s[...] = pv
    pcnt_s[...] = pc

  @pl.when(i != 0)
  def _():
    psum_s[...] = psum_s[...] + pv
    pcnt_s[...] = pcnt_s[...] + pc

  @pl.when(i == NB - 1)
  def _():
    final[...] = psum_s[...] / (jnp.float32(D) * jnp.maximum(pcnt_s[...], 1.0))


def _tc_layer_call(hm, accm, cnt, w, root, b, last, batch2=None):
  f32 = jnp.float32
  h_spec = pl.BlockSpec((BN, D), lambda i: (i, 0))
  a_spec = [pl.BlockSpec((BN, D), lambda i, r=r: (r * NB + i, 0))
            for r in range(R)]
  cnt_spec = [pl.BlockSpec((BN, 16), lambda i, r=r: (r * NB + i, 0))
              for r in range(R)]
  w_spec = pl.BlockSpec((R, D, D), lambda i: (0, 0, 0))
  root_spec = pl.BlockSpec((D, D), lambda i: (0, 0))
  b_spec = pl.BlockSpec((1, D), lambda i: (0, 0))
  params = pltpu.CompilerParams(dimension_semantics=("arbitrary",))
  if not last:
    return pl.pallas_call(
        _tc_layer0,
        grid=(NB,),
        in_specs=[h_spec] + a_spec + cnt_spec + [w_spec, root_spec, b_spec],
        out_specs=pl.BlockSpec((BN, D), lambda i: (i, 0)),
        out_shape=jax.ShapeDtypeStruct((N, D), f32),
        compiler_params=params,
    )(hm, *([accm] * R), *([cnt] * R), w, root, b)
  batch_spec = pl.BlockSpec((1, 1, BN), lambda i: (i, 0, 0))
  return pl.pallas_call(
      _tc_layer1,
      grid=(NB,),
      in_specs=[h_spec] + a_spec + cnt_spec + [w_spec, root_spec, b_spec, batch_spec],
      out_specs=pl.BlockSpec((B, 1), lambda i: (0, 0)),
      out_shape=jax.ShapeDtypeStruct((B, 1), f32),
      scratch_shapes=[pltpu.VMEM((B, 1), f32), pltpu.VMEM((B, 1), f32)],
      compiler_params=params,
  )(hm, *([accm] * R), *([cnt] * R), w, root, b, batch2)


def kernel(atom_type, edge_index, edge_type, batch, emb, W0, root0, b0, W1, root1, b1):
  i32 = jnp.int32
  src = edge_index[0].astype(i32)
  dst = edge_index[1].astype(i32)
  et = edge_type.astype(i32)

  pad_e = E_PAD - E
  ar = jnp.arange(pad_e, dtype=i32)
  srcp = jnp.concatenate([src, ar % N])
  src4 = tuple((4 * srcp + q).reshape(NS, CPT, CHUNK) for q in range(NQ))
  dstadj2 = jnp.concatenate(
      [et * N + dst, R * N + (ar % (ACC_R - R * N))]).reshape(NS, CPT, CHUNK)
  atomp = jnp.concatenate(
      [atom_type.astype(i32), jnp.arange(NH - N, dtype=i32) % VOCAB])
  atom4 = jnp.stack([4 * atomp + q for q in range(NQ)]).reshape(NQ * 4, 20, CHUNK)
  batch2 = batch.astype(i32).reshape(NB, 1, BN)

  e4 = emb.reshape(4 * VOCAB, SLAB)

  h0m, cnt = _sc_prep_call(e4, atom4, dstadj2)
  h0v = h0m.reshape(4 * NH, SLAB)
  acc0 = _sc_edge_call(h0v, src4, dstadj2)
  h1m = _tc_layer_call(h0m, acc0, cnt, W0, root0, b0.reshape(1, D), last=False)
  h1v = h1m.reshape(4 * N, SLAB)
  acc1 = _sc_edge_call(h1v, src4, dstadj2)
  final = _tc_layer_call(h1m, acc1, cnt, W1, root1, b1.reshape(1, D),
                         last=True, batch2=batch2)
  return final[:, 0]
